# baseline (speedup 1.0000x reference)
import math

import jax
import jax.numpy as jnp
from jax.experimental import pallas as pl
from jax.experimental.pallas import tpu as pltpu

_K = 2
_CF = 1.25
_MIN_CAP = 4
_E = 64
_T = 32768
_B = 2048
_NB = _T // _B
_CH = 128
_NCH = _B // _CH


def _capacity(num_tokens, num_experts):
    cap = math.floor(_K * _CF * num_tokens / num_experts)
    cap += cap % 2
    return max(cap, _MIN_CAP)

_CAP = float(_capacity(_T, _E))


def _pass1_body(logits_ref, kept0_ref, v2_ref, r1_ref, i2_ref, c0_ref,
                aux_ref, carry, me_acc):
    i = pl.program_id(0)

    @pl.when(i == 0)
    def _init():
        carry[...] = jnp.zeros_like(carry)
        me_acc[...] = jnp.zeros_like(me_acc)

    lt = logits_ref[...].T
    m = jnp.max(lt, axis=0, keepdims=True)
    ex = jnp.exp(lt - m)
    s = jnp.sum(ex, axis=0, keepdims=True)
    rs = 1.0 / s

    exi = jax.lax.bitcast_convert_type(ex, jnp.int32)
    iota_s = jax.lax.broadcasted_iota(jnp.int32, (_E, _B), 0)
    key = (exi & jnp.int32(-64)) | (63 - iota_s)
    pm = jax.lax.bitcast_convert_type(key, jnp.float32)
    v1k = jnp.max(pm, axis=0, keepdims=True)
    oh0 = (pm == v1k)
    pm2 = jnp.where(oh0, 0.0, pm)
    v2k = jnp.max(pm2, axis=0, keepdims=True)
    oh1 = (pm2 == v2k)
    v1ki = jax.lax.bitcast_convert_type(v1k, jnp.int32)
    v2ki = jax.lax.bitcast_convert_type(v2k, jnp.int32)
    i2 = 63 - (v2ki & 63)
    val1 = jax.lax.bitcast_convert_type(v1ki & jnp.int32(-64), jnp.float32) * rs
    val2 = jax.lax.bitcast_convert_type(v2ki & jnp.int32(-64), jnp.float32) * rs

    a = jnp.concatenate([oh0.astype(jnp.bfloat16),
                         oh1.astype(jnp.bfloat16)], axis=0)
    r = jax.lax.broadcasted_iota(jnp.int32, (_CH, _CH), 0)
    c = jax.lax.broadcasted_iota(jnp.int32, (_CH, _CH), 1)
    triu = (r <= c).astype(jnp.bfloat16)

    running = carry[...]
    p0_chunks = []
    p1_chunks = []
    for j in range(_NCH):
        aj = a[:, j * _CH:(j + 1) * _CH]
        cj = jnp.dot(aj, triu, preferred_element_type=jnp.float32)
        cfull = cj + running
        prod = cfull * aj.astype(jnp.float32)
        p0_chunks.append(
            jnp.sum(prod[:_E, :], axis=0, keepdims=True) - 1.0)
        p1_chunks.append(
            jnp.sum(prod[_E:, :], axis=0, keepdims=True) - 1.0)
        running = running + cj[:, _CH - 1:_CH]
    carry[...] = running
    pos0 = jnp.concatenate(p0_chunks, axis=1)
    r1 = jnp.concatenate(p1_chunks, axis=1)

    keep0 = (pos0 < _CAP).astype(jnp.float32)
    kept0_ref[...] = (val1 * keep0).reshape(1, 1, _B)
    v2_ref[...] = val2.reshape(1, 1, _B)
    r1_ref[...] = r1.reshape(1, 1, _B)
    i2_ref[...] = i2.astype(jnp.float32).reshape(1, 1, _B)

    probs = ex * rs
    pf = probs[:, :_CH]
    for j in range(1, _NCH):
        pf = pf + probs[:, j * _CH:(j + 1) * _CH]
    me = me_acc[...] + pf
    me_acc[...] = me

    @pl.when(i == _NB - 1)
    def _tail():
        new_c0 = running[:_E, :]
        c0_ref[...] = new_c0
        t = jnp.float32(_T)
        me_tot = jnp.sum(me, axis=1, keepdims=True)
        aux_ref[...] = (jnp.float32(_E) * jnp.sum(
            (me_tot / t) * (new_c0 / t))).reshape(1, 1)


def _pass2_body(kept0_ref, v2_ref, r1_ref, i2_ref, c0_ref, out0_ref, out1_ref):
    kept0 = kept0_ref[...]
    v2 = v2_ref[...]
    r1 = r1_ref[...]
    i2 = i2_ref[...]
    c0sel = jnp.zeros_like(r1)
    for e in range(_E):
        c0sel = jnp.where(i2 == float(e), c0_ref[e, 0], c0sel)
    keep1 = ((c0sel + r1) < _CAP).astype(jnp.float32)
    kv1 = v2 * keep1
    denom = kept0 + kv1 + 1e-9
    out0_ref[...] = kept0 / denom
    out1_ref[...] = kv1 / denom


@jax.jit
def kernel(logits):
    tok_spec = pl.BlockSpec((1, 1, _B), lambda i: (i, 0, 0))
    tok_shape = jax.ShapeDtypeStruct((_NB, 1, _B), jnp.float32)
    kept0, v2, r1, i2, c0, aux = pl.pallas_call(
        _pass1_body,
        grid=(_NB,),
        in_specs=[pl.BlockSpec((_B, _E), lambda i: (i, 0))],
        out_specs=[tok_spec, tok_spec, tok_spec, tok_spec,
                   pl.BlockSpec((_E, 1), lambda i: (0, 0)),
                   pl.BlockSpec((1, 1), lambda i: (0, 0))],
        out_shape=[
            tok_shape, tok_shape, tok_shape, tok_shape,
            jax.ShapeDtypeStruct((_E, 1), jnp.float32),
            jax.ShapeDtypeStruct((1, 1), jnp.float32),
        ],
        scratch_shapes=[pltpu.VMEM((2 * _E, 1), jnp.float32),
                        pltpu.VMEM((_E, _CH), jnp.float32)],
    )(logits)

    shp = (_T // 128, 128)
    out0, out1 = pl.pallas_call(
        _pass2_body,
        out_shape=[jax.ShapeDtypeStruct(shp, jnp.float32)] * 2,
    )(kept0.reshape(shp), v2.reshape(shp), r1.reshape(shp),
      i2.reshape(shp), c0)

    combine = jnp.stack([out0.reshape(-1), out1.reshape(-1)], axis=1)
    return combine, aux[0, 0]


# speedup vs baseline: 1.1089x; 1.1089x over previous
import math

import jax
import jax.numpy as jnp
from jax.experimental import pallas as pl
from jax.experimental.pallas import tpu as pltpu

_K = 2
_CF = 1.25
_MIN_CAP = 4
_E = 64
_T = 32768
_B = 2048
_NB = _T // _B
_CH = 128
_NCH = _B // _CH


def _capacity(num_tokens, num_experts):
    cap = math.floor(_K * _CF * num_tokens / num_experts)
    cap += cap % 2
    return max(cap, _MIN_CAP)

_CAP = float(_capacity(_T, _E))


def _pass1_body(logits_ref, kept0_ref, v2_ref, r1_ref, i2_ref, c0_ref,
                aux_ref, carry, me_acc):
    i = pl.program_id(0)

    @pl.when(i == 0)
    def _init():
        carry[...] = jnp.zeros_like(carry)
        me_acc[...] = jnp.zeros_like(me_acc)

    r = jax.lax.broadcasted_iota(jnp.int32, (_CH, _CH), 0)
    c = jax.lax.broadcasted_iota(jnp.int32, (_CH, _CH), 1)
    triu = (r <= c).astype(jnp.bfloat16)
    iota_s = jax.lax.broadcasted_iota(jnp.int32, (_E, _CH), 0)

    running = carry[...]
    acc = me_acc[...]
    for j in range(_NCH):
        lt = logits_ref[pl.ds(j * _CH, _CH), :].T
        m = jnp.max(lt, axis=0, keepdims=True)
        ex = jnp.exp(lt - m)
        s = jnp.sum(ex, axis=0, keepdims=True)
        rs = 1.0 / s

        exi = jax.lax.bitcast_convert_type(ex, jnp.int32)
        key = (exi & jnp.int32(-64)) | (63 - iota_s)
        pm = jax.lax.bitcast_convert_type(key, jnp.float32)
        v1k = jnp.max(pm, axis=0, keepdims=True)
        oh0 = (pm == v1k)
        pm2 = jnp.where(oh0, 0.0, pm)
        v2k = jnp.max(pm2, axis=0, keepdims=True)
        oh1 = (pm2 == v2k)
        v1ki = jax.lax.bitcast_convert_type(v1k, jnp.int32)
        v2ki = jax.lax.bitcast_convert_type(v2k, jnp.int32)
        i2 = 63 - (v2ki & 63)
        val1 = jax.lax.bitcast_convert_type(
            v1ki & jnp.int32(-64), jnp.float32) * rs
        val2 = jax.lax.bitcast_convert_type(
            v2ki & jnp.int32(-64), jnp.float32) * rs

        a = jnp.concatenate([oh0.astype(jnp.bfloat16),
                             oh1.astype(jnp.bfloat16)], axis=0)
        cj = jnp.dot(a, triu, preferred_element_type=jnp.float32)
        cfull = cj + running
        prod = cfull * a.astype(jnp.float32)
        pos0 = jnp.sum(prod[:_E, :], axis=0, keepdims=True) - 1.0
        r1 = jnp.sum(prod[_E:, :], axis=0, keepdims=True) - 1.0
        running = running + cj[:, _CH - 1:_CH]

        keep0 = (pos0 < _CAP).astype(jnp.float32)
        sl = pl.ds(j * _CH, _CH)
        kept0_ref[:, :, sl] = (val1 * keep0).reshape(1, 1, _CH)
        v2_ref[:, :, sl] = val2.reshape(1, 1, _CH)
        r1_ref[:, :, sl] = r1.reshape(1, 1, _CH)
        i2_ref[:, :, sl] = i2.astype(jnp.float32).reshape(1, 1, _CH)

        acc = acc + ex * rs

    carry[...] = running
    me_acc[...] = acc

    @pl.when(i == _NB - 1)
    def _tail():
        new_c0 = running[:_E, :]
        c0_ref[...] = new_c0
        t = jnp.float32(_T)
        me_tot = jnp.sum(acc, axis=1, keepdims=True)
        aux_ref[...] = (jnp.float32(_E) * jnp.sum(
            (me_tot / t) * (new_c0 / t))).reshape(1, 1)


def _pass2_body(kept0_ref, v2_ref, r1_ref, i2_ref, c0_ref, out0_ref, out1_ref):
    kept0 = kept0_ref[...]
    v2 = v2_ref[...]
    r1 = r1_ref[...]
    i2 = i2_ref[...]
    c0sel = jnp.zeros_like(r1)
    for e in range(_E):
        c0sel = jnp.where(i2 == float(e), c0_ref[e, 0], c0sel)
    keep1 = ((c0sel + r1) < _CAP).astype(jnp.float32)
    kv1 = v2 * keep1
    denom = kept0 + kv1 + 1e-9
    out0_ref[...] = kept0 / denom
    out1_ref[...] = kv1 / denom


@jax.jit
def kernel(logits):
    tok_spec = pl.BlockSpec((1, 1, _B), lambda i: (i, 0, 0))
    tok_shape = jax.ShapeDtypeStruct((_NB, 1, _B), jnp.float32)
    kept0, v2, r1, i2, c0, aux = pl.pallas_call(
        _pass1_body,
        grid=(_NB,),
        in_specs=[pl.BlockSpec((_B, _E), lambda i: (i, 0))],
        out_specs=[tok_spec, tok_spec, tok_spec, tok_spec,
                   pl.BlockSpec((_E, 1), lambda i: (0, 0)),
                   pl.BlockSpec((1, 1), lambda i: (0, 0))],
        out_shape=[
            tok_shape, tok_shape, tok_shape, tok_shape,
            jax.ShapeDtypeStruct((_E, 1), jnp.float32),
            jax.ShapeDtypeStruct((1, 1), jnp.float32),
        ],
        scratch_shapes=[pltpu.VMEM((2 * _E, 1), jnp.float32),
                        pltpu.VMEM((_E, _CH), jnp.float32)],
    )(logits)

    shp = (_T // 128, 128)
    out0, out1 = pl.pallas_call(
        _pass2_body,
        out_shape=[jax.ShapeDtypeStruct(shp, jnp.float32)] * 2,
    )(kept0.reshape(shp), v2.reshape(shp), r1.reshape(shp),
      i2.reshape(shp), c0)

    combine = jnp.stack([out0.reshape(-1), out1.reshape(-1)], axis=1)
    return combine, aux[0, 0]


# speedup vs baseline: 1.1346x; 1.0232x over previous
import math

import jax
import jax.numpy as jnp
from jax.experimental import pallas as pl
from jax.experimental.pallas import tpu as pltpu

_K = 2
_CF = 1.25
_MIN_CAP = 4
_E = 64
_T = 32768
_B = 4096
_NB = _T // _B
_CH = 128
_NCH = _B // _CH


def _capacity(num_tokens, num_experts):
    cap = math.floor(_K * _CF * num_tokens / num_experts)
    cap += cap % 2
    return max(cap, _MIN_CAP)

_CAP = float(_capacity(_T, _E))


def _pass1_body(logits_ref, kept0_ref, v2_ref, r1_ref, i2_ref, c0_ref,
                aux_ref, carry, me_acc):
    i = pl.program_id(0)

    @pl.when(i == 0)
    def _init():
        carry[...] = jnp.zeros_like(carry)
        me_acc[...] = jnp.zeros_like(me_acc)

    r = jax.lax.broadcasted_iota(jnp.int32, (_CH, _CH), 0)
    c = jax.lax.broadcasted_iota(jnp.int32, (_CH, _CH), 1)
    triu = (r <= c).astype(jnp.bfloat16)
    iota_s = jax.lax.broadcasted_iota(jnp.int32, (_E, _CH), 0)

    running = carry[...]
    acc = me_acc[...]
    for j in range(_NCH):
        lt = logits_ref[pl.ds(j * _CH, _CH), :].T
        m = jnp.max(lt, axis=0, keepdims=True)
        ex = jnp.exp(lt - m)
        s = jnp.sum(ex, axis=0, keepdims=True)
        rs = 1.0 / s

        exi = jax.lax.bitcast_convert_type(ex, jnp.int32)
        key = (exi & jnp.int32(-64)) | (63 - iota_s)
        pm = jax.lax.bitcast_convert_type(key, jnp.float32)
        v1k = jnp.max(pm, axis=0, keepdims=True)
        oh0 = (pm == v1k)
        pm2 = jnp.where(oh0, 0.0, pm)
        v2k = jnp.max(pm2, axis=0, keepdims=True)
        oh1 = (pm2 == v2k)
        v1ki = jax.lax.bitcast_convert_type(v1k, jnp.int32)
        v2ki = jax.lax.bitcast_convert_type(v2k, jnp.int32)
        i2 = 63 - (v2ki & 63)
        val1 = jax.lax.bitcast_convert_type(
            v1ki & jnp.int32(-64), jnp.float32) * rs
        val2 = jax.lax.bitcast_convert_type(
            v2ki & jnp.int32(-64), jnp.float32) * rs

        a = jnp.concatenate([oh0.astype(jnp.bfloat16),
                             oh1.astype(jnp.bfloat16)], axis=0)
        cj = jnp.dot(a, triu, preferred_element_type=jnp.float32)
        cfull = cj + running
        prod = cfull * a.astype(jnp.float32)
        pos0 = jnp.sum(prod[:_E, :], axis=0, keepdims=True) - 1.0
        r1 = jnp.sum(prod[_E:, :], axis=0, keepdims=True) - 1.0
        running = running + cj[:, _CH - 1:_CH]

        keep0 = (pos0 < _CAP).astype(jnp.float32)
        sl = pl.ds(j * _CH, _CH)
        kept0_ref[:, :, sl] = (val1 * keep0).reshape(1, 1, _CH)
        v2_ref[:, :, sl] = val2.reshape(1, 1, _CH)
        r1_ref[:, :, sl] = r1.reshape(1, 1, _CH)
        i2_ref[:, :, sl] = i2.astype(jnp.float32).reshape(1, 1, _CH)

        acc = acc + ex * rs

    carry[...] = running
    me_acc[...] = acc

    @pl.when(i == _NB - 1)
    def _tail():
        new_c0 = running[:_E, :]
        c0_ref[...] = new_c0
        t = jnp.float32(_T)
        me_tot = jnp.sum(acc, axis=1, keepdims=True)
        aux_ref[...] = (jnp.float32(_E) * jnp.sum(
            (me_tot / t) * (new_c0 / t))).reshape(1, 1)


def _pass2_body(kept0_ref, v2_ref, r1_ref, i2_ref, c0_ref, out0_ref, out1_ref):
    kept0 = kept0_ref[...]
    v2 = v2_ref[...]
    r1 = r1_ref[...]
    i2 = i2_ref[...]
    c0sel = jnp.zeros_like(r1)
    for e in range(_E):
        c0sel = jnp.where(i2 == float(e), c0_ref[e, 0], c0sel)
    keep1 = ((c0sel + r1) < _CAP).astype(jnp.float32)
    kv1 = v2 * keep1
    denom = kept0 + kv1 + 1e-9
    out0_ref[...] = kept0 / denom
    out1_ref[...] = kv1 / denom


@jax.jit
def kernel(logits):
    tok_spec = pl.BlockSpec((1, 1, _B), lambda i: (i, 0, 0))
    tok_shape = jax.ShapeDtypeStruct((_NB, 1, _B), jnp.float32)
    kept0, v2, r1, i2, c0, aux = pl.pallas_call(
        _pass1_body,
        grid=(_NB,),
        in_specs=[pl.BlockSpec((_B, _E), lambda i: (i, 0))],
        out_specs=[tok_spec, tok_spec, tok_spec, tok_spec,
                   pl.BlockSpec((_E, 1), lambda i: (0, 0)),
                   pl.BlockSpec((1, 1), lambda i: (0, 0))],
        out_shape=[
            tok_shape, tok_shape, tok_shape, tok_shape,
            jax.ShapeDtypeStruct((_E, 1), jnp.float32),
            jax.ShapeDtypeStruct((1, 1), jnp.float32),
        ],
        scratch_shapes=[pltpu.VMEM((2 * _E, 1), jnp.float32),
                        pltpu.VMEM((_E, _CH), jnp.float32)],
    )(logits)

    shp = (_T // 128, 128)
    out0, out1 = pl.pallas_call(
        _pass2_body,
        out_shape=[jax.ShapeDtypeStruct(shp, jnp.float32)] * 2,
    )(kept0.reshape(shp), v2.reshape(shp), r1.reshape(shp),
      i2.reshape(shp), c0)

    combine = jnp.stack([out0.reshape(-1), out1.reshape(-1)], axis=1)
    return combine, aux[0, 0]
